# Initial kernel scaffold; baseline (speedup 1.0000x reference)
import jax
import jax.numpy as jnp
from jax.experimental import pallas as pl


def kernel(x, w_t, b2):
    raise NotImplementedError("write your pallas kernel here")


# speedup vs baseline: 1.3929x; 1.3929x over previous
import jax
import jax.numpy as jnp
from jax.experimental import pallas as pl
from jax.experimental.pallas import tpu as pltpu


_TM = 1024
_TN = 1024


def _matmul_bias_tanh_kernel(x_ref, w_ref, b_ref, o_ref):
    acc = jnp.dot(x_ref[...], w_ref[...], preferred_element_type=jnp.float32)
    o_ref[...] = jnp.tanh(acc + b_ref[...])


@jax.jit
def _linear_tanh_fused(x2, w_t, b2):
    n, k = x2.shape
    kw, m = w_t.shape
    tm = min(_TM, n)
    tn = min(_TN, m)
    ni = pl.cdiv(n, tm)
    nj = pl.cdiv(m, tn)

    xb = x2.astype(jnp.bfloat16)
    wb = w_t.astype(jnp.bfloat16)

    return pl.pallas_call(
        _matmul_bias_tanh_kernel,
        out_shape=jax.ShapeDtypeStruct((n, m), jnp.float32),
        grid=(ni, nj),
        in_specs=[
            pl.BlockSpec((tm, k), lambda i, j: (i, 0)),
            pl.BlockSpec((k, tn), lambda i, j: (0, j)),
            pl.BlockSpec((1, tn), lambda i, j: (0, j)),
        ],
        out_specs=pl.BlockSpec((tm, tn), lambda i, j: (i, j)),
        compiler_params=pltpu.CompilerParams(
            dimension_semantics=("parallel", "parallel"),
        ),
    )(xb, wb, b2)


def kernel(x, w_t, b2):
    in_ch = w_t.shape[0]
    x2 = x.reshape(-1, in_ch)
    return _linear_tanh_fused(x2, w_t, b2)


# speedup vs baseline: 1.4574x; 1.0463x over previous
import jax
import jax.numpy as jnp
from jax.experimental import pallas as pl
from jax.experimental.pallas import tpu as pltpu


_TM = 1024
_TN = 512


def _matmul_bias_tanh_kernel(x_ref, w_ref, b_ref, o_ref, xb_ref):
    j = pl.program_id(1)

    @pl.when(j == 0)
    def _():
        xb_ref[...] = x_ref[...].astype(jnp.bfloat16)

    acc = jnp.dot(xb_ref[...], w_ref[...], preferred_element_type=jnp.float32)
    o_ref[...] = jnp.tanh(acc + b_ref[...])


@jax.jit
def _linear_tanh_fused(x2, w_t, b2):
    n, k = x2.shape
    m = w_t.shape[1]
    tm = min(_TM, n)
    tn = min(_TN, m)
    ni = pl.cdiv(n, tm)
    nj = pl.cdiv(m, tn)

    wb = w_t.astype(jnp.bfloat16)

    return pl.pallas_call(
        _matmul_bias_tanh_kernel,
        out_shape=jax.ShapeDtypeStruct((n, m), jnp.float32),
        grid=(ni, nj),
        in_specs=[
            pl.BlockSpec((tm, k), lambda i, j: (i, 0)),
            pl.BlockSpec((k, tn), lambda i, j: (0, j)),
            pl.BlockSpec((1, tn), lambda i, j: (0, j)),
        ],
        out_specs=pl.BlockSpec((tm, tn), lambda i, j: (i, j)),
        scratch_shapes=[pltpu.VMEM((tm, k), jnp.bfloat16)],
        compiler_params=pltpu.CompilerParams(
            dimension_semantics=("parallel", "arbitrary"),
            vmem_limit_bytes=64 * 1024 * 1024,
        ),
    )(x2, wb, b2)


def kernel(x, w_t, b2):
    in_ch = w_t.shape[0]
    x2 = x.reshape(-1, in_ch)
    return _linear_tanh_fused(x2, w_t, b2)
